# baseline (speedup 1.0000x reference)
import functools

import jax
import jax.numpy as jnp
from jax import lax
from jax.experimental import pallas as pl
from jax.experimental.pallas import tpu as pltpu
from jax.experimental.pallas import tpu_sc as plsc

N = 10000
E = 320000
D = 128
L = 3
G = 128
MAXD = 20
BN_EPS = 1e-5
QS = 256.0
QINV = 1.0 / QS

NC = 2
NS = 16
NW = NC * NS
B = 80
NCHUNK = 125
NBUF = 5
NACC = N
RPS = 624

_mesh = plsc.VectorSubcoreMesh(core_axis_name="c", subcore_axis_name="s")


def _sc_aggregate_body(h_hbm, src_hbm, dst_hbm, out_hbm,
                       src_v, dst_v, *rest):
    rows = rest[:NBUF]
    acc = rest[NBUF]
    gsems = rest[NBUF + 1:NBUF + 1 + NBUF]
    ssems = rest[NBUF + 1 + NBUF:]
    c = lax.axis_index("c")
    s = lax.axis_index("s")
    wid = c * NS + s

    def zrow(r, carry):
        for k in range(D // 32):
            rows[0][r, pl.ds(32 * k, 32)] = jnp.zeros((32,), jnp.int16)
        return carry
    lax.fori_loop(0, B, zrow, 0)
    for k in range(RPS // B):
        pltpu.sync_copy(rows[0], acc.at[pl.ds(s * RPS + k * B, B)])
    pltpu.sync_copy(rows[0].at[pl.ds(0, RPS % B)],
                    acc.at[pl.ds(s * RPS + (RPS // B) * B, RPS % B)])
    @pl.when(s == NS - 1)
    def _():
        pltpu.sync_copy(rows[0].at[pl.ds(0, 16)], acc.at[pl.ds(NS * RPS, 16)])

    pltpu.sync_copy(src_hbm.at[wid], src_v)
    pltpu.sync_copy(dst_hbm.at[wid], dst_v)
    plsc.subcore_barrier()

    def group(i, carry):
        j = NBUF * i
        ds = [pltpu.async_copy(h_hbm.at[src_v.at[j + k]], rows[k], gsems[k])
              for k in range(NBUF)]
        es = []
        for k in range(NBUF):
            ds[k].wait()
            es.append(pltpu.async_copy(rows[k], acc.at[dst_v.at[j + k]],
                                       ssems[k], add=True))
        for e in es:
            e.wait()
        return carry
    lax.fori_loop(0, NCHUNK // NBUF, group, 0)
    plsc.subcore_barrier()

    pltpu.sync_copy(acc.at[pl.ds(s * RPS, RPS)],
                    out_hbm.at[c, pl.ds(s * RPS, RPS)])
    @pl.when(s == NS - 1)
    def _():
        pltpu.sync_copy(acc.at[pl.ds(NS * RPS, 16)],
                        out_hbm.at[c, pl.ds(NS * RPS, 16)])


_sc_aggregate = pl.kernel(
    _sc_aggregate_body,
    out_type=jax.ShapeDtypeStruct((NC, N, D), jnp.int16),
    mesh=_mesh,
    scratch_types=(
        [pltpu.VMEM((NCHUNK, B), jnp.int32),
         pltpu.VMEM((NCHUNK, B), jnp.int32)]
        + [pltpu.VMEM((B, D), jnp.int16)] * NBUF
        + [pltpu.VMEM_SHARED((NACC, D), jnp.int16)]
        + [pltpu.SemaphoreType.DMA] * (2 * NBUF)
    ),
    compiler_params=pltpu.CompilerParams(use_tc_tiling_on_sc=False),
)


RB = 2000


def _enc_body(feat_ref, depth_ref, w_ref, b_ref, demb_ref, out_ref):
    d = jnp.clip(depth_ref[...], 0, MAXD - 1)
    oh = (d == lax.broadcasted_iota(jnp.int32, (RB, 128), 1)).astype(jnp.float32)
    h = jnp.dot(feat_ref[...], w_ref[...], preferred_element_type=jnp.float32)
    h = h + b_ref[...]
    h = h + jnp.dot(oh, demb_ref[...], preferred_element_type=jnp.float32)
    out_ref[...] = jnp.floor(h * QS + 0.5).astype(jnp.int16)


def _encoder(feat, depth2, w, b2, demb_pad):
    return pl.pallas_call(
        _enc_body,
        grid=(N // RB,),
        in_specs=[
            pl.BlockSpec((RB, D), lambda i: (i, 0)),
            pl.BlockSpec((RB, 1), lambda i: (i, 0)),
            pl.BlockSpec((D, D), lambda i: (0, 0)),
            pl.BlockSpec((1, D), lambda i: (0, 0)),
            pl.BlockSpec((128, D), lambda i: (0, 0)),
        ],
        out_specs=pl.BlockSpec((RB, D), lambda i: (i, 0)),
        out_shape=jax.ShapeDtypeStruct((N, D), jnp.int16),
    )(feat, depth2, w, b2, demb_pad)


def _bn_core(h_ref, a0_ref, a1_ref, eps_ref, g_ref, bt_ref):
    h2 = (h_ref[...].astype(jnp.float32) * (1.0 + eps_ref[0, 0])
          + a0_ref[...].astype(jnp.float32)
          + a1_ref[...].astype(jnp.float32)) * QINV
    mean = jnp.mean(h2, axis=0, keepdims=True)
    var = jnp.mean(h2 * h2, axis=0, keepdims=True) - mean * mean
    return (h2 - mean) * lax.rsqrt(var + BN_EPS) * g_ref[...] + bt_ref[...]


def _bn_body(h_ref, a0_ref, a1_ref, eps_ref, g_ref, bt_ref, out_ref):
    y = _bn_core(h_ref, a0_ref, a1_ref, eps_ref, g_ref, bt_ref)
    out_ref[...] = jnp.floor(jnp.maximum(y, 0.0) * QS + 0.5).astype(jnp.int16)


def _bn(h, parts, eps_l, g2, bt2):
    return pl.pallas_call(
        _bn_body,
        in_specs=[
            pl.BlockSpec((N, D), lambda: (0, 0)),
            pl.BlockSpec((N, D), lambda: (0, 0)),
            pl.BlockSpec((N, D), lambda: (0, 0)),
            pl.BlockSpec((1, 1), lambda: (0, 0)),
            pl.BlockSpec((1, D), lambda: (0, 0)),
            pl.BlockSpec((1, D), lambda: (0, 0)),
        ],
        out_specs=pl.BlockSpec((N, D), lambda: (0, 0)),
        out_shape=jax.ShapeDtypeStruct((N, D), jnp.int16),
    )(h, parts[0], parts[1], eps_l, g2, bt2)


def _bn_pool_body(h_ref, a0_ref, a1_ref, eps_ref, g_ref, bt_ref, gid_ref, out_ref):
    y = _bn_core(h_ref, a0_ref, a1_ref, eps_ref, g_ref, bt_ref)
    oh = (gid_ref[...] == lax.broadcasted_iota(jnp.int32, (N, G), 1)).astype(jnp.float32)
    out_ref[...] = lax.dot_general(oh, y, (((0,), (0,)), ((), ())),
                                   preferred_element_type=jnp.float32)


def _bn_pool(h, parts, eps_l, g2, bt2, gid2):
    return pl.pallas_call(
        _bn_pool_body,
        in_specs=[
            pl.BlockSpec((N, D), lambda: (0, 0)),
            pl.BlockSpec((N, D), lambda: (0, 0)),
            pl.BlockSpec((N, D), lambda: (0, 0)),
            pl.BlockSpec((1, 1), lambda: (0, 0)),
            pl.BlockSpec((1, D), lambda: (0, 0)),
            pl.BlockSpec((1, D), lambda: (0, 0)),
            pl.BlockSpec((N, 1), lambda: (0, 0)),
        ],
        out_specs=pl.BlockSpec((G, D), lambda: (0, 0)),
        out_shape=jax.ShapeDtypeStruct((G, D), jnp.float32),
    )(h, parts[0], parts[1], eps_l, g2, bt2, gid2)


def kernel(feat, depth, edge_index, graph_ids, W_enc, b_enc, depth_emb,
           eps, gammas, betas):
    depth2 = depth.reshape(N, 1).astype(jnp.int32)
    demb_pad = jnp.zeros((128, D), jnp.float32).at[:MAXD].set(depth_emb)
    b2 = b_enc.reshape(1, D)
    gid2 = graph_ids.reshape(N, 1).astype(jnp.int32)
    src3 = edge_index[0].reshape(NW, NCHUNK, B).astype(jnp.int32)
    dst3 = edge_index[1].reshape(NW, NCHUNK, B).astype(jnp.int32)

    h = _encoder(feat, depth2, W_enc, b2, demb_pad)
    out = None
    for layer in range(L):
        parts = _sc_aggregate(h, src3, dst3)
        eps_l = eps[layer].reshape(1, 1)
        g2 = gammas[layer].reshape(1, D)
        bt2 = betas[layer].reshape(1, D)
        if layer < L - 1:
            h = _bn(h, parts, eps_l, g2, bt2)
        else:
            out = _bn_pool(h, parts, eps_l, g2, bt2, gid2)
    return out
